# baseline (speedup 1.0000x reference)
import functools

import jax
import jax.numpy as jnp
from jax import lax
from jax.experimental import pallas as pl
from jax.experimental.pallas import tpu as pltpu
from jax.experimental.pallas import tpu_sc as plsc

N1, E1 = 10000, 160000
N2, E2 = 2500, 40000
N3, E3 = 625, 10000
N4 = 160
K = 5
K3 = K * K * K
NUM_CLASSES = 4

N1P, N2P, N3P, N4P = 10240, 2560, 640, 160
E1P, E2P, E3P = 160768, 40960, 10240

_MESH = plsc.VectorSubcoreMesh(core_axis_name="c", subcore_axis_name="s")
_SC_PARAMS = pltpu.CompilerParams(use_tc_tiling_on_sc=False)
_BITS = [(b & 1, (b >> 1) & 1, (b >> 2) & 1) for b in range(8)]


def _basis(u8, ei8, n_src_pad):
    ep = u8.shape[0]
    be = 512

    def body(u_ref, e_ref, w_ref, r_ref, k_ref):
        pos0 = u_ref[:, 0:1] * (K - 1)
        pos1 = u_ref[:, 1:2] * (K - 1)
        pos2 = u_ref[:, 2:3] * (K - 1)
        valid = u_ref[:, 3:4]
        lo0 = jnp.clip(jnp.floor(pos0), 0.0, float(K - 2))
        lo1 = jnp.clip(jnp.floor(pos1), 0.0, float(K - 2))
        lo2 = jnp.clip(jnp.floor(pos2), 0.0, float(K - 2))
        f0, f1, f2 = pos0 - lo0, pos1 - lo1, pos2 - lo2
        l0 = lo0.astype(jnp.int32)
        l1 = lo1.astype(jnp.int32)
        l2 = lo2.astype(jnp.int32)
        src = e_ref[:, 0:1]
        for b, (b0, b1, b2) in enumerate(_BITS):
            w = (f0 if b0 else 1.0 - f0) * (f1 if b1 else 1.0 - f1)
            w = w * (f2 if b2 else 1.0 - f2) * valid
            k = (l0 + b0) + K * (l1 + b1) + K * K * (l2 + b2)
            w_ref[:, b:b + 1] = w
            r_ref[:, b:b + 1] = k * n_src_pad + src
            k_ref[:, b:b + 1] = k

    return pl.pallas_call(
        body,
        grid=(ep // be,),
        in_specs=[
            pl.BlockSpec((be, 8), lambda i: (i, 0)),
            pl.BlockSpec((be, 8), lambda i: (i, 0)),
        ],
        out_specs=[
            pl.BlockSpec((be, 8), lambda i: (i, 0)),
            pl.BlockSpec((be, 8), lambda i: (i, 0)),
            pl.BlockSpec((be, 8), lambda i: (i, 0)),
        ],
        out_shape=[
            jax.ShapeDtypeStruct((ep, 8), jnp.float32),
            jax.ShapeDtypeStruct((ep, 8), jnp.int32),
            jax.ShapeDtypeStruct((ep, 8), jnp.int32),
        ],
    )(u8, ei8)


def _project(x, w3):
    np_, in_dim = x.shape
    d = w3.shape[2]
    br = {10240: 1024, 2560: 512, 640: 640, 160: 160}[np_]
    kb = 25

    def body(x_ref, w_ref, o_ref):
        xb = x_ref[...]
        for t in range(kb):
            o_ref[t] = jnp.dot(xb, w_ref[t], preferred_element_type=jnp.float32)

    return pl.pallas_call(
        body,
        grid=(K3 // kb, np_ // br),
        in_specs=[
            pl.BlockSpec((br, in_dim), lambda ki, ni: (ni, 0)),
            pl.BlockSpec((kb, in_dim, d), lambda ki, ni: (ki, 0, 0)),
        ],
        out_specs=pl.BlockSpec((kb, br, d), lambda ki, ni: (ki, ni, 0)),
        out_shape=jax.ShapeDtypeStruct((K3, np_, d), jnp.float32),
    )(x, w3)


def _edge_aggregate(xp_flat, r8f, w8f, dstp, ep, nout_p, d):
    dw = d + 16
    ept = ep // 32
    nchunks = ept // 16
    npairs = nchunks // 2
    assert nchunks % 2 == 0
    rpt = nout_p // 16
    zr = 40
    assert rpt % zr == 0

    @functools.partial(
        pl.kernel,
        out_type=jax.ShapeDtypeStruct((2, nout_p, dw), jnp.float32),
        mesh=_MESH,
        compiler_params=_SC_PARAMS,
        scratch_types=[
            pltpu.VMEM((128,), jnp.int32),
            pltpu.VMEM((128,), jnp.int32),
            pltpu.VMEM((128,), jnp.float32),
            pltpu.VMEM((128,), jnp.float32),
            pltpu.VMEM((16,), jnp.int32),
            pltpu.VMEM((16,), jnp.int32),
            pltpu.VMEM((128, d), jnp.float32),
            pltpu.VMEM((128, d), jnp.float32),
            pltpu.VMEM((16, dw), jnp.float32),
            pltpu.VMEM((16,), jnp.float32),
            pltpu.VMEM((zr, dw), jnp.float32),
            pltpu.VMEM_SHARED((nout_p, dw), jnp.float32),
            pltpu.SemaphoreType.DMA,
            pltpu.SemaphoreType.DMA,
            pltpu.SemaphoreType.DMA,
            pltpu.SemaphoreType.DMA,
        ],
    )
    def agg(xp_hbm, r_hbm, w_hbm, dst_hbm, out_hbm,
            idx_a, idx_b, w_a, w_b, dst_a, dst_b, rows_a, rows_b,
            msg_v, one_v, z_v, acc_sh, sm_a, sm_b, sg_a, sg_b):
        cid = lax.axis_index("c")
        sid = lax.axis_index("s")
        wid = cid * 16 + sid
        one_v[pl.ds(0, 16)] = jnp.ones((16,), jnp.float32)

        bufs = [(idx_a, w_a, dst_a, rows_a, sm_a, sg_a),
                (idx_b, w_b, dst_b, rows_b, sm_b, sg_b)]

        def start_meta(p, ebase):
            idx_v, w_v, dst_v, _, sm, _ = bufs[p]
            pltpu.async_copy(r_hbm.at[pl.ds(ebase * 8, 128)], idx_v, sm)
            pltpu.async_copy(w_hbm.at[pl.ds(ebase * 8, 128)], w_v, sm)
            pltpu.async_copy(dst_hbm.at[pl.ds(ebase, 16)], dst_v, sm)

        def wait_meta(p):
            idx_v, w_v, dst_v, _, sm, _ = bufs[p]
            pltpu.make_async_copy(r_hbm.at[pl.ds(0, 128)], idx_v, sm).wait()
            pltpu.make_async_copy(w_hbm.at[pl.ds(0, 128)], w_v, sm).wait()
            pltpu.make_async_copy(dst_hbm.at[pl.ds(0, 16)], dst_v, sm).wait()

        def start_gather(p):
            idx_v, _, _, rows_v, _, sg = bufs[p]
            return pltpu.async_copy(xp_hbm.at[idx_v], rows_v, sg)

        def compute_scatter(p):
            _, w_v, dst_v, rows_v, _, _ = bufs[p]
            one16 = one_v[pl.ds(0, 16)]
            for i2 in range(8):
                wv = w_v[pl.ds(i2 * 16, 16)]
                for half in range(2):
                    i = i2 * 2 + half
                    for j in range(d // 16):
                        acc = wv[half * 8] * rows_v[i * 8, pl.ds(16 * j, 16)]
                        for c in range(1, 8):
                            acc = acc + wv[half * 8 + c] * rows_v[
                                i * 8 + c, pl.ds(16 * j, 16)]
                        msg_v[i, pl.ds(16 * j, 16)] = acc
                    dacc = wv[half * 8] * one16
                    for c in range(1, 8):
                        dacc = dacc + wv[half * 8 + c] * one16
                    msg_v[i, pl.ds(d, 16)] = dacc
            pltpu.sync_copy(msg_v, acc_sh.at[dst_v], add=True)

        def zinit(t, _):
            for j in range(dw // 16):
                z_v[t, pl.ds(16 * j, 16)] = jnp.zeros((16,), jnp.float32)
            return _
        lax.fori_loop(0, zr, zinit, None)
        rbase = sid * rpt

        def zcopy(t, _):
            pltpu.sync_copy(z_v, acc_sh.at[pl.ds(rbase + t * zr, zr)])
            return _
        lax.fori_loop(0, rpt // zr, zcopy, None)
        plsc.subcore_barrier()

        tbase = wid * ept
        start_meta(0, tbase)
        start_meta(1, tbase + 16)

        def pair(gg, _):
            eb_a = tbase + gg * 32
            wait_meta(0)
            g_a = start_gather(0)
            wait_meta(1)
            g_b = start_gather(1)
            g_a.wait()
            compute_scatter(0)

            @pl.when(gg + 1 < npairs)
            def _():
                start_meta(0, eb_a + 32)
            g_b.wait()
            compute_scatter(1)

            @pl.when(gg + 1 < npairs)
            def _():
                start_meta(1, eb_a + 48)
            return _
        lax.fori_loop(0, npairs, pair, None)
        plsc.subcore_barrier()
        pltpu.sync_copy(acc_sh.at[pl.ds(rbase, rpt)],
                        out_hbm.at[cid, pl.ds(rbase, rpt)])

    return agg(xp_flat, r8f, w8f, dstp)


def _gather_x(xflat, srcp):
    gp = srcp.shape[0]
    per_tile = gp // 32
    rg = 32
    nchunks = per_tile // rg

    @functools.partial(
        pl.kernel,
        out_type=jax.ShapeDtypeStruct((gp,), jnp.float32),
        mesh=_MESH,
        compiler_params=_SC_PARAMS,
        scratch_types=[
            pltpu.VMEM((rg,), jnp.int32),
            pltpu.VMEM((rg,), jnp.int32),
            pltpu.VMEM((rg,), jnp.float32),
            pltpu.VMEM((rg,), jnp.float32),
            pltpu.SemaphoreType.DMA,
            pltpu.SemaphoreType.DMA,
            pltpu.SemaphoreType.DMA,
            pltpu.SemaphoreType.DMA,
        ],
    )
    def gat(x_hbm, s_hbm, out_hbm, i_a, i_b, v_a, v_b, sm_a, sm_b, sg_a, sg_b):
        cid = lax.axis_index("c")
        sid = lax.axis_index("s")
        wid = cid * 16 + sid
        tb = wid * per_tile
        bufs = [(i_a, v_a, sm_a, sg_a), (i_b, v_b, sm_b, sg_b)]

        def start_idx(p, g):
            i_v, _, sm, _ = bufs[p]
            pltpu.async_copy(s_hbm.at[pl.ds(tb + g * rg, rg)], i_v, sm)

        def wait_idx(p):
            i_v, _, sm, _ = bufs[p]
            pltpu.make_async_copy(s_hbm.at[pl.ds(0, rg)], i_v, sm).wait()

        start_idx(0, 0)
        start_idx(1, 1)

        def pair(gg, _):
            for p in range(2):
                i_v, v_v, _, sg = bufs[p]
                wait_idx(p)
                pltpu.async_copy(x_hbm.at[i_v], v_v, sg).wait()
                pltpu.sync_copy(v_v, out_hbm.at[pl.ds(tb + (gg * 2 + p) * rg, rg)])

                @pl.when(gg + 1 < nchunks // 2)
                def _():
                    start_idx(p, gg * 2 + 2 + p)
            return _
        lax.fori_loop(0, nchunks // 2, pair, None)

    return gat(xflat, srcp)


def _scale_w8(w8, xg):
    ep = w8.shape[0]
    be = 512

    def body(w_ref, x_ref, o_ref):
        o_ref[...] = w_ref[...] * x_ref[...]

    return pl.pallas_call(
        body,
        grid=(ep // be,),
        in_specs=[
            pl.BlockSpec((be, 8), lambda i: (i, 0)),
            pl.BlockSpec((be, 1), lambda i: (i, 0)),
        ],
        out_specs=pl.BlockSpec((be, 8), lambda i: (i, 0)),
        out_shape=jax.ShapeDtypeStruct((ep, 8), jnp.float32),
    )(w8, xg)


def _edge_aggregate_x1(wrow, k8f, wx8f, w8f, dstp, ep, nout_p):
    d = wrow.shape[1]
    dw = d + 16
    ept = ep // 32
    nchunks = ept // 16
    npairs = nchunks // 2
    assert nchunks % 2 == 0
    rpt = nout_p // 16
    zr = 40
    assert rpt % zr == 0

    @functools.partial(
        pl.kernel,
        out_type=jax.ShapeDtypeStruct((2, nout_p, dw), jnp.float32),
        mesh=_MESH,
        compiler_params=_SC_PARAMS,
        scratch_types=[
            pltpu.VMEM((128,), jnp.int32),
            pltpu.VMEM((128,), jnp.int32),
            pltpu.VMEM((128,), jnp.float32),
            pltpu.VMEM((128,), jnp.float32),
            pltpu.VMEM((128,), jnp.float32),
            pltpu.VMEM((128,), jnp.float32),
            pltpu.VMEM((16,), jnp.int32),
            pltpu.VMEM((16,), jnp.int32),
            pltpu.VMEM((128, d), jnp.float32),
            pltpu.VMEM((128, d), jnp.float32),
            pltpu.VMEM((16, dw), jnp.float32),
            pltpu.VMEM((16,), jnp.float32),
            pltpu.VMEM((zr, dw), jnp.float32),
            pltpu.VMEM_SHARED((nout_p, dw), jnp.float32),
            pltpu.SemaphoreType.DMA,
            pltpu.SemaphoreType.DMA,
            pltpu.SemaphoreType.DMA,
            pltpu.SemaphoreType.DMA,
        ],
    )
    def agg(t_hbm, k_hbm, wx_hbm, w_hbm, dst_hbm, out_hbm,
            idx_a, idx_b, wx_a, wx_b, w_a, w_b, dst_a, dst_b,
            rows_a, rows_b, msg_v, one_v, z_v, acc_sh,
            sm_a, sm_b, sg_a, sg_b):
        cid = lax.axis_index("c")
        sid = lax.axis_index("s")
        wid = cid * 16 + sid
        one_v[pl.ds(0, 16)] = jnp.ones((16,), jnp.float32)

        bufs = [(idx_a, wx_a, w_a, dst_a, rows_a, sm_a, sg_a),
                (idx_b, wx_b, w_b, dst_b, rows_b, sm_b, sg_b)]

        def start_meta(p, ebase):
            idx_v, wx_v, w_v, dst_v, _, sm, _ = bufs[p]
            pltpu.async_copy(k_hbm.at[pl.ds(ebase * 8, 128)], idx_v, sm)
            pltpu.async_copy(wx_hbm.at[pl.ds(ebase * 8, 128)], wx_v, sm)
            pltpu.async_copy(w_hbm.at[pl.ds(ebase * 8, 128)], w_v, sm)
            pltpu.async_copy(dst_hbm.at[pl.ds(ebase, 16)], dst_v, sm)

        def wait_meta(p):
            idx_v, wx_v, w_v, dst_v, _, sm, _ = bufs[p]
            pltpu.make_async_copy(k_hbm.at[pl.ds(0, 128)], idx_v, sm).wait()
            pltpu.make_async_copy(wx_hbm.at[pl.ds(0, 128)], wx_v, sm).wait()
            pltpu.make_async_copy(w_hbm.at[pl.ds(0, 128)], w_v, sm).wait()
            pltpu.make_async_copy(dst_hbm.at[pl.ds(0, 16)], dst_v, sm).wait()

        def start_gather(p):
            idx_v, _, _, _, rows_v, _, sg = bufs[p]
            return pltpu.async_copy(t_hbm.at[idx_v], rows_v, sg)

        def compute_scatter(p):
            _, wx_v, w_v, dst_v, rows_v, _, _ = bufs[p]
            one16 = one_v[pl.ds(0, 16)]
            for i2 in range(8):
                wxv = wx_v[pl.ds(i2 * 16, 16)]
                wv = w_v[pl.ds(i2 * 16, 16)]
                for half in range(2):
                    i = i2 * 2 + half
                    for j in range(d // 16):
                        acc = wxv[half * 8] * rows_v[i * 8, pl.ds(16 * j, 16)]
                        for c in range(1, 8):
                            acc = acc + wxv[half * 8 + c] * rows_v[
                                i * 8 + c, pl.ds(16 * j, 16)]
                        msg_v[i, pl.ds(16 * j, 16)] = acc
                    dacc = wv[half * 8] * one16
                    for c in range(1, 8):
                        dacc = dacc + wv[half * 8 + c] * one16
                    msg_v[i, pl.ds(d, 16)] = dacc
            pltpu.sync_copy(msg_v, acc_sh.at[dst_v], add=True)

        def zinit(t, _):
            for j in range(dw // 16):
                z_v[t, pl.ds(16 * j, 16)] = jnp.zeros((16,), jnp.float32)
            return _
        lax.fori_loop(0, zr, zinit, None)
        rbase = sid * rpt

        def zcopy(t, _):
            pltpu.sync_copy(z_v, acc_sh.at[pl.ds(rbase + t * zr, zr)])
            return _
        lax.fori_loop(0, rpt // zr, zcopy, None)
        plsc.subcore_barrier()

        tbase = wid * ept
        start_meta(0, tbase)
        start_meta(1, tbase + 16)

        def pair(gg, _):
            eb_a = tbase + gg * 32
            wait_meta(0)
            g_a = start_gather(0)
            wait_meta(1)
            g_b = start_gather(1)
            g_a.wait()
            compute_scatter(0)

            @pl.when(gg + 1 < npairs)
            def _():
                start_meta(0, eb_a + 32)
            g_b.wait()
            compute_scatter(1)

            @pl.when(gg + 1 < npairs)
            def _():
                start_meta(1, eb_a + 48)
            return _
        lax.fori_loop(0, npairs, pair, None)
        plsc.subcore_barrier()
        pltpu.sync_copy(acc_sh.at[pl.ds(rbase, rpt)],
                        out_hbm.at[cid, pl.ds(rbase, rpt)])

    return agg(wrow, k8f, wx8f, w8f, dstp)


def _pool_max_sc(xin, clp, nout_p):
    nin_p, d = xin.shape
    own = nout_p // 32
    cb = 128 if (nin_p // 128) % 2 == 0 and nin_p % 128 == 0 else 64
    nchunks = nin_p // cb
    npairs = nchunks // 2
    assert nchunks % 2 == 0

    @functools.partial(
        pl.kernel,
        out_type=jax.ShapeDtypeStruct((nout_p, d), jnp.float32),
        mesh=_MESH,
        compiler_params=_SC_PARAMS,
        scratch_types=[
            pltpu.VMEM((cb, d), jnp.float32),
            pltpu.VMEM((cb, d), jnp.float32),
            pltpu.VMEM((cb,), jnp.int32),
            pltpu.VMEM((cb,), jnp.int32),
            pltpu.VMEM((own, d), jnp.float32),
            pltpu.SemaphoreType.DMA,
            pltpu.SemaphoreType.DMA,
        ],
    )
    def pool(x_hbm, cl_hbm, out_hbm, x_a, x_b, cl_a, cl_b, acc_v, sm_a, sm_b):
        cid = lax.axis_index("c")
        sid = lax.axis_index("s")
        wid = cid * 16 + sid
        lo = wid * own
        bufs = [(x_a, cl_a, sm_a), (x_b, cl_b, sm_b)]

        def start(p, g):
            x_v, cl_v, sm = bufs[p]
            pltpu.async_copy(x_hbm.at[pl.ds(g * cb, cb)], x_v, sm)
            pltpu.async_copy(cl_hbm.at[pl.ds(g * cb, cb)], cl_v, sm)

        def wait(p):
            x_v, cl_v, sm = bufs[p]
            pltpu.make_async_copy(x_hbm.at[pl.ds(0, cb)], x_v, sm).wait()
            pltpu.make_async_copy(cl_hbm.at[pl.ds(0, cb)], cl_v, sm).wait()

        def scan(p):
            x_v, cl_v, _ = bufs[p]
            for i2 in range(cb // 16):
                cvec = cl_v[pl.ds(i2 * 16, 16)] - lo
                for i in range(16):
                    rel = cvec[i]

                    @pl.when(jnp.logical_and(rel >= 0, rel < own))
                    def _():
                        for j in range(d // 16):
                            cur = acc_v[rel, pl.ds(16 * j, 16)]
                            acc_v[rel, pl.ds(16 * j, 16)] = jnp.maximum(
                                cur, x_v[i2 * 16 + i, pl.ds(16 * j, 16)])

        neg = jnp.full((16,), -jnp.inf, jnp.float32)
        for r in range(own):
            for j in range(d // 16):
                acc_v[r, pl.ds(16 * j, 16)] = neg

        start(0, 0)
        start(1, 1)

        def pair(gg, _):
            wait(0)
            scan(0)

            @pl.when(gg + 1 < npairs)
            def _():
                start(0, gg * 2 + 2)
            wait(1)
            scan(1)

            @pl.when(gg + 1 < npairs)
            def _():
                start(1, gg * 2 + 3)
            return _
        lax.fori_loop(0, npairs, pair, None)

        for r in range(own):
            for j in range(d // 16):
                v = acc_v[r, pl.ds(16 * j, 16)]
                acc_v[r, pl.ds(16 * j, 16)] = jnp.where(
                    v == -jnp.inf, jnp.zeros((16,), jnp.float32), v)
        pltpu.sync_copy(acc_v, out_hbm.at[pl.ds(lo, own)])

    return pool(xin, clp)


def _gather_rows(table, idxp, ntiles, rg):
    gp = idxp.shape[0]
    d = table.shape[1]
    per_tile = gp // ntiles
    nchunks = per_tile // rg

    @functools.partial(
        pl.kernel,
        out_type=jax.ShapeDtypeStruct((gp, d), jnp.float32),
        mesh=_MESH,
        compiler_params=_SC_PARAMS,
        scratch_types=[
            pltpu.VMEM((rg,), jnp.int32),
            pltpu.VMEM((rg, d), jnp.float32),
            pltpu.SemaphoreType.DMA,
        ],
    )
    def gat(t_hbm, i_hbm, out_hbm, idx_v, rows_v, sem):
        cid = lax.axis_index("c")
        sid = lax.axis_index("s")
        wid = cid * 16 + sid

        @pl.when(wid < ntiles)
        def _():
            def chunk(g, _):
                base = wid * per_tile + g * rg
                pltpu.sync_copy(i_hbm.at[pl.ds(base, rg)], idx_v)
                pltpu.async_copy(t_hbm.at[idx_v], rows_v, sem).wait()
                pltpu.sync_copy(rows_v, out_hbm.at[pl.ds(base, rg)])
                return _
            lax.fori_loop(0, nchunks, chunk, None)

    return gat(table, idxp)


def _combine(aggp, x, wr, b):
    np_, in_dim = x.shape
    d = wr.shape[1]
    dw = aggp.shape[2]
    br = {10240: 1024, 2560: 512, 640: 640, 160: 160}[np_]

    def body(a_ref, x_ref, w_ref, b_ref, o_ref):
        a = a_ref[0] + a_ref[1]
        agg = a[:, :d]
        deg = a[:, d:d + 1]
        s = agg / jnp.maximum(deg, 1.0)
        y = s + jnp.dot(x_ref[...], w_ref[...],
                        preferred_element_type=jnp.float32) + b_ref[...]
        o_ref[...] = jnp.where(y > 0, y, jnp.exp(jnp.minimum(y, 0.0)) - 1.0)

    return pl.pallas_call(
        body,
        grid=(np_ // br,),
        in_specs=[
            pl.BlockSpec((2, br, dw), lambda i: (0, i, 0)),
            pl.BlockSpec((br, in_dim), lambda i: (i, 0)),
            pl.BlockSpec((in_dim, d), lambda i: (0, 0)),
            pl.BlockSpec((1, d), lambda i: (0, 0)),
        ],
        out_specs=pl.BlockSpec((br, d), lambda i: (i, 0)),
        out_shape=jax.ShapeDtypeStruct((np_, d), jnp.float32),
    )(aggp, x, wr, b.reshape(1, d))


def _dense(x, w, b, elu):
    np_, in_dim = x.shape
    d = w.shape[1]
    br = {10240: 1024, 2560: 512, 640: 640, 160: 160}[np_]

    def body(x_ref, w_ref, b_ref, o_ref):
        y = jnp.dot(x_ref[...], w_ref[...],
                    preferred_element_type=jnp.float32) + b_ref[...]
        if elu:
            y = jnp.where(y > 0, y, jnp.exp(jnp.minimum(y, 0.0)) - 1.0)
        o_ref[...] = y

    return pl.pallas_call(
        body,
        grid=(np_ // br,),
        in_specs=[
            pl.BlockSpec((br, in_dim), lambda i: (i, 0)),
            pl.BlockSpec((in_dim, d), lambda i: (0, 0)),
            pl.BlockSpec((1, d), lambda i: (0, 0)),
        ],
        out_specs=pl.BlockSpec((br, d), lambda i: (i, 0)),
        out_shape=jax.ShapeDtypeStruct((np_, d), jnp.float32),
    )(x, w, b.reshape(1, d))


def _final(x, w, b):
    np_, in_dim = x.shape
    d = w.shape[1]
    br = 1024

    def body(x_ref, w_ref, b_ref, o_ref):
        y = jnp.dot(x_ref[...], w_ref[...],
                    preferred_element_type=jnp.float32) + b_ref[...]
        m = jnp.max(y, axis=1, keepdims=True)
        lse = jnp.log(jnp.sum(jnp.exp(y - m), axis=1, keepdims=True)) + m
        o_ref[...] = y - lse

    return pl.pallas_call(
        body,
        grid=(np_ // br,),
        in_specs=[
            pl.BlockSpec((br, in_dim), lambda i: (i, 0)),
            pl.BlockSpec((in_dim, d), lambda i: (0, 0)),
            pl.BlockSpec((1, d), lambda i: (0, 0)),
        ],
        out_specs=pl.BlockSpec((br, d), lambda i: (i, 0)),
        out_shape=jax.ShapeDtypeStruct((np_, d), jnp.float32),
    )(x, w, b.reshape(1, d))


def _edge_setup(edge_index, edge_attr, ep, n_src_pad):
    e = edge_index.shape[1]
    u8 = jnp.zeros((ep, 8), jnp.float32)
    u8 = u8.at[:e, 0:3].set(edge_attr)
    u8 = u8.at[:e, 3].set(1.0)
    ei8 = jnp.zeros((ep, 8), jnp.int32)
    ei8 = ei8.at[:e, 0].set(edge_index[0])
    dstp = jnp.zeros((ep,), jnp.int32).at[:e].set(edge_index[1])
    srcp = jnp.zeros((ep,), jnp.int32).at[:e].set(edge_index[0])
    w8, r8, k8 = _basis(u8, ei8, n_src_pad)
    return w8.reshape(-1), r8.reshape(-1), k8.reshape(-1), dstp, srcp


def _cluster_pads(c, nin_p):
    n = c.shape[0]
    c_pool = jnp.full((nin_p,), 1 << 30, jnp.int32).at[:n].set(c)
    c_gath = jnp.zeros((nin_p,), jnp.int32).at[:n].set(c)
    return c_pool, c_gath


def _spline_layer(x, w3, wr, b, r8f, w8f, dstp, ep, nout_p):
    xp = _project(x, w3).reshape(K3 * x.shape[0], w3.shape[2])
    aggp = _edge_aggregate(xp, r8f, w8f, dstp, ep, nout_p, w3.shape[2])
    return _combine(aggp, x, wr, b)


def kernel(x, edge_index1, edge_attr1, edge_index2, edge_attr2, edge_index3,
           edge_attr3, c1, c2, c3, W1, Wr1, b1, W2, Wr2, b2, W4, Wr4, b4,
           W5, Wr5, b5, W6, Wr6, b6, fc1_w, fc1_b, fc2_w, fc2_b,
           skip1_w, skip1_b, skip2_w, skip2_b, skip3_w, skip3_b):
    xp0 = jnp.zeros((N1P, 1), jnp.float32).at[:N1].set(x)

    w8_1, r8_1, k8_1, dst1, src1 = _edge_setup(edge_index1, edge_attr1, E1P, N1P)
    w8_2, r8_2, _, dst2, _ = _edge_setup(edge_index2, edge_attr2, E2P, N2P)
    w8_3, r8_3, _, dst3, _ = _edge_setup(edge_index3, edge_attr3, E3P, N3P)
    c1_pool, c1_gath = _cluster_pads(c1, N1P)
    c2_pool, c2_gath = _cluster_pads(c2, N2P)
    c3_pool, c3_gath = _cluster_pads(c3, N3P)

    xg1 = _gather_x(xp0.reshape(-1), src1)
    w8x1 = _scale_w8(w8_1.reshape(-1, 8), xg1.reshape(-1, 1)).reshape(-1)
    agg1 = _edge_aggregate_x1(W1[:, 0, :], k8_1, w8x1, w8_1, dst1, E1P, N1P)
    x1 = _combine(agg1, xp0, Wr1, b1)
    p2 = _pool_max_sc(x1, c1_pool, N2P)
    x2 = _spline_layer(p2, W2, Wr2, b2, r8_2, w8_2, dst2, E2P, N2P)
    p3 = _pool_max_sc(x2, c2_pool, N3P)
    x3 = _spline_layer(p3, W2, Wr2, b2, r8_3, w8_3, dst3, E3P, N3P)
    p4 = _pool_max_sc(x3, c3_pool, N4P)
    x4 = _dense(p4, fc1_w, fc1_b, elu=True)

    g3 = _gather_rows(x4, c3_gath, 16, 40)
    s3 = _dense(x3, skip3_w, skip3_b, elu=False)
    cat3 = jnp.concatenate([g3, s3], axis=1)
    x3b = _spline_layer(cat3, W4, Wr4, b4, r8_3, w8_3, dst3, E3P, N3P)

    g2 = _gather_rows(x3b, c2_gath, 32, 80)
    s2 = _dense(x2, skip2_w, skip2_b, elu=False)
    cat2 = jnp.concatenate([g2, s2], axis=1)
    x2b = _spline_layer(cat2, W5, Wr5, b5, r8_2, w8_2, dst2, E2P, N2P)

    g1 = _gather_rows(x2b, c1_gath, 32, 64)
    s1 = _dense(x1, skip1_w, skip1_b, elu=False)
    cat1 = jnp.concatenate([g1, s1], axis=1)
    x1b = _spline_layer(cat1, W6, Wr6, b6, r8_1, w8_1, dst1, E1P, N1P)

    out = _final(x1b, fc2_w, fc2_b)
    return out[:N1]


# speedup vs baseline: 1.1480x; 1.1480x over previous
import functools

import jax
import jax.numpy as jnp
from jax import lax
from jax.experimental import pallas as pl
from jax.experimental.pallas import tpu as pltpu
from jax.experimental.pallas import tpu_sc as plsc

N1, E1 = 10000, 160000
N2, E2 = 2500, 40000
N3, E3 = 625, 10000
N4 = 160
K = 5
K3 = K * K * K
NUM_CLASSES = 4

N1P, N2P, N3P, N4P = 10240, 2560, 640, 160
E1P, E2P, E3P = 160768, 40960, 10240

_MESH = plsc.VectorSubcoreMesh(core_axis_name="c", subcore_axis_name="s")
_SC_PARAMS = pltpu.CompilerParams(use_tc_tiling_on_sc=False)
_BITS = [(b & 1, (b >> 1) & 1, (b >> 2) & 1) for b in range(8)]


def _basis(u8, ei8, n_src_pad):
    ep = u8.shape[0]
    be = 512

    def body(u_ref, e_ref, w_ref, r_ref, k_ref):
        pos0 = u_ref[:, 0:1] * (K - 1)
        pos1 = u_ref[:, 1:2] * (K - 1)
        pos2 = u_ref[:, 2:3] * (K - 1)
        valid = u_ref[:, 3:4]
        lo0 = jnp.clip(jnp.floor(pos0), 0.0, float(K - 2))
        lo1 = jnp.clip(jnp.floor(pos1), 0.0, float(K - 2))
        lo2 = jnp.clip(jnp.floor(pos2), 0.0, float(K - 2))
        f0, f1, f2 = pos0 - lo0, pos1 - lo1, pos2 - lo2
        l0 = lo0.astype(jnp.int32)
        l1 = lo1.astype(jnp.int32)
        l2 = lo2.astype(jnp.int32)
        src = e_ref[:, 0:1]
        for b, (b0, b1, b2) in enumerate(_BITS):
            w = (f0 if b0 else 1.0 - f0) * (f1 if b1 else 1.0 - f1)
            w = w * (f2 if b2 else 1.0 - f2) * valid
            k = (l0 + b0) + K * (l1 + b1) + K * K * (l2 + b2)
            w_ref[:, b:b + 1] = w
            r_ref[:, b:b + 1] = k * n_src_pad + src
            k_ref[:, b:b + 1] = k

    return pl.pallas_call(
        body,
        grid=(ep // be,),
        in_specs=[
            pl.BlockSpec((be, 8), lambda i: (i, 0)),
            pl.BlockSpec((be, 8), lambda i: (i, 0)),
        ],
        out_specs=[
            pl.BlockSpec((be, 8), lambda i: (i, 0)),
            pl.BlockSpec((be, 8), lambda i: (i, 0)),
            pl.BlockSpec((be, 8), lambda i: (i, 0)),
        ],
        out_shape=[
            jax.ShapeDtypeStruct((ep, 8), jnp.float32),
            jax.ShapeDtypeStruct((ep, 8), jnp.int32),
            jax.ShapeDtypeStruct((ep, 8), jnp.int32),
        ],
    )(u8, ei8)


def _project(x, w3):
    np_, in_dim = x.shape
    d = w3.shape[2]
    br = {10240: 1024, 2560: 512, 640: 640, 160: 160}[np_]
    kb = 25

    def body(x_ref, w_ref, o_ref):
        xb = x_ref[...]
        for t in range(kb):
            o_ref[t] = jnp.dot(xb, w_ref[t], preferred_element_type=jnp.float32)

    return pl.pallas_call(
        body,
        grid=(K3 // kb, np_ // br),
        in_specs=[
            pl.BlockSpec((br, in_dim), lambda ki, ni: (ni, 0)),
            pl.BlockSpec((kb, in_dim, d), lambda ki, ni: (ki, 0, 0)),
        ],
        out_specs=pl.BlockSpec((kb, br, d), lambda ki, ni: (ki, ni, 0)),
        out_shape=jax.ShapeDtypeStruct((K3, np_, d), jnp.float32),
    )(x, w3)


def _edge_aggregate(xp_flat, r8f, w8f, dstp, ep, nout_p, d):
    dw = d + 16
    ept = ep // 32
    nchunks = ept // 16
    npairs = nchunks // 2
    assert nchunks % 2 == 0
    rpt = nout_p // 16
    zr = 40
    assert rpt % zr == 0

    @functools.partial(
        pl.kernel,
        out_type=jax.ShapeDtypeStruct((2, nout_p, dw), jnp.float32),
        mesh=_MESH,
        compiler_params=_SC_PARAMS,
        scratch_types=[
            pltpu.VMEM((128,), jnp.int32),
            pltpu.VMEM((128,), jnp.int32),
            pltpu.VMEM((128,), jnp.float32),
            pltpu.VMEM((128,), jnp.float32),
            pltpu.VMEM((16,), jnp.int32),
            pltpu.VMEM((16,), jnp.int32),
            pltpu.VMEM((128, d), jnp.float32),
            pltpu.VMEM((128, d), jnp.float32),
            pltpu.VMEM((16, dw), jnp.float32),
            pltpu.VMEM((16,), jnp.float32),
            pltpu.VMEM((zr, dw), jnp.float32),
            pltpu.VMEM_SHARED((nout_p, dw), jnp.float32),
            pltpu.SemaphoreType.DMA,
            pltpu.SemaphoreType.DMA,
            pltpu.SemaphoreType.DMA,
            pltpu.SemaphoreType.DMA,
        ],
    )
    def agg(xp_hbm, r_hbm, w_hbm, dst_hbm, out_hbm,
            idx_a, idx_b, w_a, w_b, dst_a, dst_b, rows_a, rows_b,
            msg_v, one_v, z_v, acc_sh, sm_a, sm_b, sg_a, sg_b):
        cid = lax.axis_index("c")
        sid = lax.axis_index("s")
        wid = cid * 16 + sid
        one_v[pl.ds(0, 16)] = jnp.ones((16,), jnp.float32)

        bufs = [(idx_a, w_a, dst_a, rows_a, sm_a, sg_a),
                (idx_b, w_b, dst_b, rows_b, sm_b, sg_b)]

        def start_meta(p, ebase):
            idx_v, w_v, dst_v, _, sm, _ = bufs[p]
            pltpu.async_copy(r_hbm.at[pl.ds(ebase * 8, 128)], idx_v, sm)
            pltpu.async_copy(w_hbm.at[pl.ds(ebase * 8, 128)], w_v, sm)
            pltpu.async_copy(dst_hbm.at[pl.ds(ebase, 16)], dst_v, sm)

        def wait_meta(p):
            idx_v, w_v, dst_v, _, sm, _ = bufs[p]
            pltpu.make_async_copy(r_hbm.at[pl.ds(0, 128)], idx_v, sm).wait()
            pltpu.make_async_copy(w_hbm.at[pl.ds(0, 128)], w_v, sm).wait()
            pltpu.make_async_copy(dst_hbm.at[pl.ds(0, 16)], dst_v, sm).wait()

        def start_gather(p):
            idx_v, _, _, rows_v, _, sg = bufs[p]
            return pltpu.async_copy(xp_hbm.at[idx_v], rows_v, sg)

        def compute_scatter(p):
            _, w_v, dst_v, rows_v, _, _ = bufs[p]
            one16 = one_v[pl.ds(0, 16)]
            for i2 in range(8):
                wv = w_v[pl.ds(i2 * 16, 16)]
                for half in range(2):
                    i = i2 * 2 + half
                    for j in range(d // 16):
                        acc = wv[half * 8] * rows_v[i * 8, pl.ds(16 * j, 16)]
                        for c in range(1, 8):
                            acc = acc + wv[half * 8 + c] * rows_v[
                                i * 8 + c, pl.ds(16 * j, 16)]
                        msg_v[i, pl.ds(16 * j, 16)] = acc
                    dacc = wv[half * 8] * one16
                    for c in range(1, 8):
                        dacc = dacc + wv[half * 8 + c] * one16
                    msg_v[i, pl.ds(d, 16)] = dacc
            pltpu.sync_copy(msg_v, acc_sh.at[dst_v], add=True)

        def zinit(t, _):
            for j in range(dw // 16):
                z_v[t, pl.ds(16 * j, 16)] = jnp.zeros((16,), jnp.float32)
            return _
        lax.fori_loop(0, zr, zinit, None)
        rbase = sid * rpt

        def zcopy(t, _):
            pltpu.sync_copy(z_v, acc_sh.at[pl.ds(rbase + t * zr, zr)])
            return _
        lax.fori_loop(0, rpt // zr, zcopy, None)
        plsc.subcore_barrier()

        tbase = wid * ept
        start_meta(0, tbase)
        start_meta(1, tbase + 16)

        def pair(gg, _):
            eb_a = tbase + gg * 32
            wait_meta(0)
            g_a = start_gather(0)
            wait_meta(1)
            g_b = start_gather(1)
            g_a.wait()
            compute_scatter(0)

            @pl.when(gg + 1 < npairs)
            def _():
                start_meta(0, eb_a + 32)
            g_b.wait()
            compute_scatter(1)

            @pl.when(gg + 1 < npairs)
            def _():
                start_meta(1, eb_a + 48)
            return _
        lax.fori_loop(0, npairs, pair, None)
        plsc.subcore_barrier()
        pltpu.sync_copy(acc_sh.at[pl.ds(rbase, rpt)],
                        out_hbm.at[cid, pl.ds(rbase, rpt)])

    return agg(xp_flat, r8f, w8f, dstp)


def _edge_aggregate_scalar(xflat, wrow, k8f, w8f, srcp, dstp, ep, nout_p):
    d = wrow.shape[1]
    dw = d + 16
    ept = ep // 32
    nchunks = ept // 16
    rpt = nout_p // 16
    zr = 40
    assert rpt % zr == 0

    @functools.partial(
        pl.kernel,
        out_type=jax.ShapeDtypeStruct((2, nout_p, dw), jnp.float32),
        mesh=_MESH,
        compiler_params=_SC_PARAMS,
        scratch_types=[
            pltpu.VMEM((16,), jnp.float32),
            pltpu.VMEM((K3, d), jnp.float32),
            pltpu.VMEM((128,), jnp.int32),
            pltpu.VMEM((128,), jnp.float32),
            pltpu.VMEM((16,), jnp.int32),
            pltpu.VMEM((16,), jnp.int32),
            pltpu.VMEM((16,), jnp.float32),
            pltpu.VMEM((16, dw), jnp.float32),
            pltpu.VMEM((zr, dw), jnp.float32),
            pltpu.VMEM_SHARED((nout_p, dw), jnp.float32),
            pltpu.SemaphoreType.DMA,
            pltpu.SemaphoreType.DMA,
            pltpu.SemaphoreType.DMA,
            pltpu.SemaphoreType.DMA,
        ],
    )
    def agg(x_hbm, t_hbm, k_hbm, w_hbm, src_hbm, dst_hbm, out_hbm,
            xs_v, t_v, k_v, w_v, src_v, dst_v, one_v, msg_v, z_v, acc_sh,
            sem, sem2, sem3, sem4):
        cid = lax.axis_index("c")
        sid = lax.axis_index("s")
        wid = cid * 16 + sid
        one_v[pl.ds(0, 16)] = jnp.ones((16,), jnp.float32)
        pltpu.sync_copy(t_hbm, t_v)

        def zinit(t, _):
            for j in range(dw // 16):
                z_v[t, pl.ds(16 * j, 16)] = jnp.zeros((16,), jnp.float32)
            return _
        lax.fori_loop(0, zr, zinit, None)
        rbase = sid * rpt

        def zcopy(t, _):
            pltpu.sync_copy(z_v, acc_sh.at[pl.ds(rbase + t * zr, zr)])
            return _
        lax.fori_loop(0, rpt // zr, zcopy, None)
        plsc.subcore_barrier()

        tbase = wid * ept

        def chunk(g, _):
            ebase = tbase + g * 16
            cp1 = pltpu.async_copy(k_hbm.at[pl.ds(ebase * 8, 128)], k_v, sem)
            cp2 = pltpu.async_copy(w_hbm.at[pl.ds(ebase * 8, 128)], w_v, sem2)
            cp3 = pltpu.async_copy(src_hbm.at[pl.ds(ebase, 16)], src_v, sem3)
            cp4 = pltpu.async_copy(dst_hbm.at[pl.ds(ebase, 16)], dst_v, sem4)
            cp1.wait()
            cp2.wait()
            cp3.wait()
            cp4.wait()
            pltpu.async_copy(x_hbm.at[src_v], xs_v, sem3).wait()
            xs = xs_v[pl.ds(0, 16)]
            one16 = one_v[pl.ds(0, 16)]
            for i2 in range(8):
                wv = w_v[pl.ds(i2 * 16, 16)]
                kv = k_v[pl.ds(i2 * 16, 16)]
                for half in range(2):
                    i = i2 * 2 + half
                    xi = xs[i]
                    for j in range(d // 16):
                        acc = wv[half * 8] * t_v[kv[half * 8], pl.ds(16 * j, 16)]
                        for c in range(1, 8):
                            acc = acc + wv[half * 8 + c] * t_v[
                                kv[half * 8 + c], pl.ds(16 * j, 16)]
                        msg_v[i, pl.ds(16 * j, 16)] = xi * acc
                    dacc = wv[half * 8] * one16
                    for c in range(1, 8):
                        dacc = dacc + wv[half * 8 + c] * one16
                    msg_v[i, pl.ds(d, 16)] = dacc
            pltpu.sync_copy(msg_v, acc_sh.at[dst_v], add=True)
            return _
        lax.fori_loop(0, nchunks, chunk, None)
        plsc.subcore_barrier()
        pltpu.sync_copy(acc_sh.at[pl.ds(rbase, rpt)],
                        out_hbm.at[cid, pl.ds(rbase, rpt)])

    return agg(xflat, wrow, k8f, w8f, srcp, dstp)


def _pool_max_sc(xin, clp, nout_p):
    nin_p, d = xin.shape
    own = nout_p // 32
    cb = 128 if (nin_p // 128) % 2 == 0 and nin_p % 128 == 0 else 64
    nchunks = nin_p // cb
    npairs = nchunks // 2
    assert nchunks % 2 == 0

    @functools.partial(
        pl.kernel,
        out_type=jax.ShapeDtypeStruct((nout_p, d), jnp.float32),
        mesh=_MESH,
        compiler_params=_SC_PARAMS,
        scratch_types=[
            pltpu.VMEM((cb, d), jnp.float32),
            pltpu.VMEM((cb, d), jnp.float32),
            pltpu.VMEM((cb,), jnp.int32),
            pltpu.VMEM((cb,), jnp.int32),
            pltpu.VMEM((own, d), jnp.float32),
            pltpu.SemaphoreType.DMA,
            pltpu.SemaphoreType.DMA,
        ],
    )
    def pool(x_hbm, cl_hbm, out_hbm, x_a, x_b, cl_a, cl_b, acc_v, sm_a, sm_b):
        cid = lax.axis_index("c")
        sid = lax.axis_index("s")
        wid = cid * 16 + sid
        lo = wid * own
        bufs = [(x_a, cl_a, sm_a), (x_b, cl_b, sm_b)]

        def start(p, g):
            x_v, cl_v, sm = bufs[p]
            pltpu.async_copy(x_hbm.at[pl.ds(g * cb, cb)], x_v, sm)
            pltpu.async_copy(cl_hbm.at[pl.ds(g * cb, cb)], cl_v, sm)

        def wait(p):
            x_v, cl_v, sm = bufs[p]
            pltpu.make_async_copy(x_hbm.at[pl.ds(0, cb)], x_v, sm).wait()
            pltpu.make_async_copy(cl_hbm.at[pl.ds(0, cb)], cl_v, sm).wait()

        def scan(p):
            x_v, cl_v, _ = bufs[p]
            for i2 in range(cb // 16):
                cvec = cl_v[pl.ds(i2 * 16, 16)] - lo
                for i in range(16):
                    rel = cvec[i]

                    @pl.when(jnp.logical_and(rel >= 0, rel < own))
                    def _():
                        for j in range(d // 16):
                            cur = acc_v[rel, pl.ds(16 * j, 16)]
                            acc_v[rel, pl.ds(16 * j, 16)] = jnp.maximum(
                                cur, x_v[i2 * 16 + i, pl.ds(16 * j, 16)])

        neg = jnp.full((16,), -jnp.inf, jnp.float32)
        for r in range(own):
            for j in range(d // 16):
                acc_v[r, pl.ds(16 * j, 16)] = neg

        start(0, 0)
        start(1, 1)

        def pair(gg, _):
            wait(0)
            scan(0)

            @pl.when(gg + 1 < npairs)
            def _():
                start(0, gg * 2 + 2)
            wait(1)
            scan(1)

            @pl.when(gg + 1 < npairs)
            def _():
                start(1, gg * 2 + 3)
            return _
        lax.fori_loop(0, npairs, pair, None)

        for r in range(own):
            for j in range(d // 16):
                v = acc_v[r, pl.ds(16 * j, 16)]
                acc_v[r, pl.ds(16 * j, 16)] = jnp.where(
                    v == -jnp.inf, jnp.zeros((16,), jnp.float32), v)
        pltpu.sync_copy(acc_v, out_hbm.at[pl.ds(lo, own)])

    return pool(xin, clp)


def _gather_rows(table, idxp, ntiles, rg):
    gp = idxp.shape[0]
    d = table.shape[1]
    per_tile = gp // ntiles
    nchunks = per_tile // rg

    @functools.partial(
        pl.kernel,
        out_type=jax.ShapeDtypeStruct((gp, d), jnp.float32),
        mesh=_MESH,
        compiler_params=_SC_PARAMS,
        scratch_types=[
            pltpu.VMEM((rg,), jnp.int32),
            pltpu.VMEM((rg, d), jnp.float32),
            pltpu.SemaphoreType.DMA,
        ],
    )
    def gat(t_hbm, i_hbm, out_hbm, idx_v, rows_v, sem):
        cid = lax.axis_index("c")
        sid = lax.axis_index("s")
        wid = cid * 16 + sid

        @pl.when(wid < ntiles)
        def _():
            def chunk(g, _):
                base = wid * per_tile + g * rg
                pltpu.sync_copy(i_hbm.at[pl.ds(base, rg)], idx_v)
                pltpu.async_copy(t_hbm.at[idx_v], rows_v, sem).wait()
                pltpu.sync_copy(rows_v, out_hbm.at[pl.ds(base, rg)])
                return _
            lax.fori_loop(0, nchunks, chunk, None)

    return gat(table, idxp)


def _combine(aggp, x, wr, b):
    np_, in_dim = x.shape
    d = wr.shape[1]
    dw = aggp.shape[2]
    br = {10240: 1024, 2560: 512, 640: 640, 160: 160}[np_]

    def body(a_ref, x_ref, w_ref, b_ref, o_ref):
        a = a_ref[0] + a_ref[1]
        agg = a[:, :d]
        deg = a[:, d:d + 1]
        s = agg / jnp.maximum(deg, 1.0)
        y = s + jnp.dot(x_ref[...], w_ref[...],
                        preferred_element_type=jnp.float32) + b_ref[...]
        o_ref[...] = jnp.where(y > 0, y, jnp.exp(jnp.minimum(y, 0.0)) - 1.0)

    return pl.pallas_call(
        body,
        grid=(np_ // br,),
        in_specs=[
            pl.BlockSpec((2, br, dw), lambda i: (0, i, 0)),
            pl.BlockSpec((br, in_dim), lambda i: (i, 0)),
            pl.BlockSpec((in_dim, d), lambda i: (0, 0)),
            pl.BlockSpec((1, d), lambda i: (0, 0)),
        ],
        out_specs=pl.BlockSpec((br, d), lambda i: (i, 0)),
        out_shape=jax.ShapeDtypeStruct((np_, d), jnp.float32),
    )(aggp, x, wr, b.reshape(1, d))


def _dense(x, w, b, elu):
    np_, in_dim = x.shape
    d = w.shape[1]
    br = {10240: 1024, 2560: 512, 640: 640, 160: 160}[np_]

    def body(x_ref, w_ref, b_ref, o_ref):
        y = jnp.dot(x_ref[...], w_ref[...],
                    preferred_element_type=jnp.float32) + b_ref[...]
        if elu:
            y = jnp.where(y > 0, y, jnp.exp(jnp.minimum(y, 0.0)) - 1.0)
        o_ref[...] = y

    return pl.pallas_call(
        body,
        grid=(np_ // br,),
        in_specs=[
            pl.BlockSpec((br, in_dim), lambda i: (i, 0)),
            pl.BlockSpec((in_dim, d), lambda i: (0, 0)),
            pl.BlockSpec((1, d), lambda i: (0, 0)),
        ],
        out_specs=pl.BlockSpec((br, d), lambda i: (i, 0)),
        out_shape=jax.ShapeDtypeStruct((np_, d), jnp.float32),
    )(x, w, b.reshape(1, d))


def _final(x, w, b):
    np_, in_dim = x.shape
    d = w.shape[1]
    br = 1024

    def body(x_ref, w_ref, b_ref, o_ref):
        y = jnp.dot(x_ref[...], w_ref[...],
                    preferred_element_type=jnp.float32) + b_ref[...]
        m = jnp.max(y, axis=1, keepdims=True)
        lse = jnp.log(jnp.sum(jnp.exp(y - m), axis=1, keepdims=True)) + m
        o_ref[...] = y - lse

    return pl.pallas_call(
        body,
        grid=(np_ // br,),
        in_specs=[
            pl.BlockSpec((br, in_dim), lambda i: (i, 0)),
            pl.BlockSpec((in_dim, d), lambda i: (0, 0)),
            pl.BlockSpec((1, d), lambda i: (0, 0)),
        ],
        out_specs=pl.BlockSpec((br, d), lambda i: (i, 0)),
        out_shape=jax.ShapeDtypeStruct((np_, d), jnp.float32),
    )(x, w, b.reshape(1, d))


def _edge_setup(edge_index, edge_attr, ep, n_src_pad):
    e = edge_index.shape[1]
    u8 = jnp.zeros((ep, 8), jnp.float32)
    u8 = u8.at[:e, 0:3].set(edge_attr)
    u8 = u8.at[:e, 3].set(1.0)
    ei8 = jnp.zeros((ep, 8), jnp.int32)
    ei8 = ei8.at[:e, 0].set(edge_index[0])
    dstp = jnp.zeros((ep,), jnp.int32).at[:e].set(edge_index[1])
    srcp = jnp.zeros((ep,), jnp.int32).at[:e].set(edge_index[0])
    w8, r8, k8 = _basis(u8, ei8, n_src_pad)
    return w8.reshape(-1), r8.reshape(-1), k8.reshape(-1), dstp, srcp


def _cluster_pads(c, nin_p):
    n = c.shape[0]
    c_pool = jnp.full((nin_p,), 1 << 30, jnp.int32).at[:n].set(c)
    c_gath = jnp.zeros((nin_p,), jnp.int32).at[:n].set(c)
    return c_pool, c_gath


def _spline_layer(x, w3, wr, b, r8f, w8f, dstp, ep, nout_p):
    xp = _project(x, w3).reshape(K3 * x.shape[0], w3.shape[2])
    aggp = _edge_aggregate(xp, r8f, w8f, dstp, ep, nout_p, w3.shape[2])
    return _combine(aggp, x, wr, b)


def kernel(x, edge_index1, edge_attr1, edge_index2, edge_attr2, edge_index3,
           edge_attr3, c1, c2, c3, W1, Wr1, b1, W2, Wr2, b2, W4, Wr4, b4,
           W5, Wr5, b5, W6, Wr6, b6, fc1_w, fc1_b, fc2_w, fc2_b,
           skip1_w, skip1_b, skip2_w, skip2_b, skip3_w, skip3_b):
    xp0 = jnp.zeros((N1P, 1), jnp.float32).at[:N1].set(x)

    w8_1, r8_1, k8_1, dst1, src1 = _edge_setup(edge_index1, edge_attr1, E1P, N1P)
    w8_2, r8_2, _, dst2, _ = _edge_setup(edge_index2, edge_attr2, E2P, N2P)
    w8_3, r8_3, _, dst3, _ = _edge_setup(edge_index3, edge_attr3, E3P, N3P)
    c1_pool, c1_gath = _cluster_pads(c1, N1P)
    c2_pool, c2_gath = _cluster_pads(c2, N2P)
    c3_pool, c3_gath = _cluster_pads(c3, N3P)

    agg1 = _edge_aggregate_scalar(xp0.reshape(-1), W1[:, 0, :], k8_1, w8_1,
                                  src1, dst1, E1P, N1P)
    x1 = _combine(agg1, xp0, Wr1, b1)
    p2 = _pool_max_sc(x1, c1_pool, N2P)
    x2 = _spline_layer(p2, W2, Wr2, b2, r8_2, w8_2, dst2, E2P, N2P)
    p3 = _pool_max_sc(x2, c2_pool, N3P)
    x3 = _spline_layer(p3, W2, Wr2, b2, r8_3, w8_3, dst3, E3P, N3P)
    p4 = _pool_max_sc(x3, c3_pool, N4P)
    x4 = _dense(p4, fc1_w, fc1_b, elu=True)

    g3 = _gather_rows(x4, c3_gath, 16, 40)
    s3 = _dense(x3, skip3_w, skip3_b, elu=False)
    cat3 = jnp.concatenate([g3, s3], axis=1)
    x3b = _spline_layer(cat3, W4, Wr4, b4, r8_3, w8_3, dst3, E3P, N3P)

    g2 = _gather_rows(x3b, c2_gath, 32, 80)
    s2 = _dense(x2, skip2_w, skip2_b, elu=False)
    cat2 = jnp.concatenate([g2, s2], axis=1)
    x2b = _spline_layer(cat2, W5, Wr5, b5, r8_2, w8_2, dst2, E2P, N2P)

    g1 = _gather_rows(x2b, c1_gath, 32, 64)
    s1 = _dense(x1, skip1_w, skip1_b, elu=False)
    cat1 = jnp.concatenate([g1, s1], axis=1)
    x1b = _spline_layer(cat1, W6, Wr6, b6, r8_1, w8_1, dst1, E1P, N1P)

    out = _final(x1b, fc2_w, fc2_b)
    return out[:N1]
